# baseline (speedup 1.0000x reference)
import jax
import jax.numpy as jnp
from jax import lax
from jax.experimental import pallas as pl
from jax.experimental.pallas import tpu as pltpu, tpu_sc as plsc

B, N, M, C, K = 8, 65536, 4096, 64, 32
CH = C + 3
NT = B * CH

NC, NS, L = 2, 16, 16
NW = NC * NS
TASKS_PER_W = (NT + NW - 1) // NW

RM = 64
NCHUNK = M // RM
JPR = K // L


def _qag_kernel(xyz_t, nxyz_t, feat, idx3, nf, gx,
                table_v, idx_v0, idx_v1, outa_v0, outa_v1,
                outb_v0, outb_v1, cen_v,
                sem_t, sem_c, sem_i0, sem_i1, sem_o0, sem_o1, sem_b0, sem_b1):
    cid = lax.axis_index("c")
    sid = lax.axis_index("s")
    wid = sid * NC + cid

    idx_v = (idx_v0, idx_v1)
    outa_v = (outa_v0, outa_v1)
    outb_v = (outb_v0, outb_v1)
    sem_i = (sem_i0, sem_i1)
    sem_o = (sem_o0, sem_o1)
    sem_b = (sem_b0, sem_b1)

    def gather_chunk(iv_ref, oa_ref):
        @plsc.parallel_loop(0, RM, 1, unroll=8)
        def _(r):
            for j in range(JPR):
                iv = iv_ref[r, pl.ds(j * L, L)]
                oa_ref[r, pl.ds(j * L, L)] = plsc.load_gather(table_v, [iv])

    def gather_diff_chunk(iv_ref, oa_ref, ob_ref, h):
        m_base = h * RM

        def _body(r, carry):
            mi = lax.broadcast(m_base + r, (L,))
            cen = plsc.load_gather(cen_v, [mi])
            for j in range(JPR):
                iv = iv_ref[r, pl.ds(j * L, L)]
                raw = plsc.load_gather(table_v, [iv])
                oa_ref[r, pl.ds(j * L, L)] = raw
                ob_ref[r, pl.ds(j * L, L)] = raw - cen
            return carry
        lax.fori_loop(0, RM, _body, 0, unroll=False)

    def run_task(ti, carry):
        t = wid + ti * NW

        @pl.when(t < NT)
        def _():
            b = t // CH
            ch = t - b * CH
            is_xyz = ch < 3

            def idx_src(h):
                return idx3.at[b, pl.ds(h * RM, RM), :]

            @pl.when(jnp.logical_not(is_xyz))
            def _feat():
                tc = pltpu.async_copy(feat.at[b, ch - 3], table_v, sem_t)
                pltpu.async_copy(idx_src(0), idx_v[0], sem_i[0])
                tc.wait()
                for h in range(NCHUNK):
                    p = h % 2
                    if h + 1 < NCHUNK:
                        pltpu.async_copy(idx_src(h + 1), idx_v[(h + 1) % 2],
                                         sem_i[(h + 1) % 2])
                    pltpu.make_async_copy(idx_src(h), idx_v[p], sem_i[p]).wait()
                    if h >= 2:
                        pltpu.make_async_copy(
                            outa_v[p],
                            nf.at[b, ch, pl.ds((h - 2) * RM, RM), :],
                            sem_o[p]).wait()
                    gather_chunk(idx_v[p], outa_v[p])
                    pltpu.async_copy(
                        outa_v[p], nf.at[b, ch, pl.ds(h * RM, RM), :],
                        sem_o[p])
                for h in (NCHUNK - 2, NCHUNK - 1):
                    p = h % 2
                    pltpu.make_async_copy(
                        outa_v[p], nf.at[b, ch, pl.ds(h * RM, RM), :],
                        sem_o[p]).wait()

            @pl.when(is_xyz)
            def _xyz():
                tc = pltpu.async_copy(xyz_t.at[b, ch], table_v, sem_t)
                cc = pltpu.async_copy(nxyz_t.at[b, ch], cen_v, sem_c)
                pltpu.async_copy(idx_src(0), idx_v[0], sem_i[0])
                tc.wait()
                cc.wait()
                for h in range(NCHUNK):
                    p = h % 2
                    if h + 1 < NCHUNK:
                        pltpu.async_copy(idx_src(h + 1), idx_v[(h + 1) % 2],
                                         sem_i[(h + 1) % 2])
                    pltpu.make_async_copy(idx_src(h), idx_v[p], sem_i[p]).wait()
                    if h >= 2:
                        pltpu.make_async_copy(
                            outa_v[p],
                            gx.at[b, ch, pl.ds((h - 2) * RM, RM), :],
                            sem_o[p]).wait()
                        pltpu.make_async_copy(
                            outb_v[p],
                            nf.at[b, ch, pl.ds((h - 2) * RM, RM), :],
                            sem_b[p]).wait()
                    gather_diff_chunk(idx_v[p], outa_v[p], outb_v[p], h)
                    pltpu.async_copy(
                        outa_v[p], gx.at[b, ch, pl.ds(h * RM, RM), :],
                        sem_o[p])
                    pltpu.async_copy(
                        outb_v[p], nf.at[b, ch, pl.ds(h * RM, RM), :],
                        sem_b[p])
                for h in (NCHUNK - 2, NCHUNK - 1):
                    p = h % 2
                    pltpu.make_async_copy(
                        outa_v[p], gx.at[b, ch, pl.ds(h * RM, RM), :],
                        sem_o[p]).wait()
                    pltpu.make_async_copy(
                        outb_v[p], nf.at[b, ch, pl.ds(h * RM, RM), :],
                        sem_b[p]).wait()

        return carry

    lax.fori_loop(0, TASKS_PER_W, run_task, 0, unroll=False)


@jax.jit
def kernel(xyz, new_xyz, features, idx):
    xyz_t = jnp.transpose(xyz, (0, 2, 1))
    nxyz_t = jnp.transpose(new_xyz, (0, 2, 1))
    idx3 = idx.astype(jnp.int32)

    mesh = plsc.VectorSubcoreMesh(core_axis_name="c", subcore_axis_name="s")
    new_features, grouped_xyz = pl.kernel(
        _qag_kernel,
        out_type=(
            jax.ShapeDtypeStruct((B, CH, M, K), jnp.float32),
            jax.ShapeDtypeStruct((B, 3, M, K), jnp.float32),
        ),
        mesh=mesh,
        scratch_types=[
            pltpu.VMEM((N,), jnp.float32),
            pltpu.VMEM((RM, K), jnp.int32),
            pltpu.VMEM((RM, K), jnp.int32),
            pltpu.VMEM((RM, K), jnp.float32),
            pltpu.VMEM((RM, K), jnp.float32),
            pltpu.VMEM((RM, K), jnp.float32),
            pltpu.VMEM((RM, K), jnp.float32),
            pltpu.VMEM((M,), jnp.float32),
            pltpu.SemaphoreType.DMA,
            pltpu.SemaphoreType.DMA,
            pltpu.SemaphoreType.DMA,
            pltpu.SemaphoreType.DMA,
            pltpu.SemaphoreType.DMA,
            pltpu.SemaphoreType.DMA,
            pltpu.SemaphoreType.DMA,
            pltpu.SemaphoreType.DMA,
        ],
        compiler_params=pltpu.CompilerParams(needs_layout_passes=False),
    )(xyz_t, nxyz_t, features, idx3)

    return new_features, grouped_xyz


# speedup vs baseline: 1.4321x; 1.4321x over previous
import jax
import jax.numpy as jnp
from jax import lax
from jax.experimental import pallas as pl
from jax.experimental.pallas import tpu as pltpu, tpu_sc as plsc

B, N, M, C, K = 8, 65536, 4096, 64, 32
MK = M * K
CH = C + 3
NT = B * CH

NC, NS, L = 2, 16, 16
NW = NC * NS
TASKS_PER_W = (NT + NW - 1) // NW

CE = 8192
NCHUNK = MK // CE
NV = CE // L
KV = K // L


def _qag_kernel(xyz_t, nxyz_t, feat, idx3, nf, gx,
                table_v, idx_v0, idx_v1, outa_v0, outa_v1,
                outb_v0, outb_v1, cen_v,
                sem_t, sem_c, sem_i0, sem_i1, sem_o0, sem_o1, sem_b0, sem_b1):
    cid = lax.axis_index("c")
    sid = lax.axis_index("s")
    wid = sid * NC + cid

    idx_v = (idx_v0, idx_v1)
    outa_v = (outa_v0, outa_v1)
    outb_v = (outb_v0, outb_v1)
    sem_i = (sem_i0, sem_i1)
    sem_o = (sem_o0, sem_o1)
    sem_b = (sem_b0, sem_b1)

    def gather_chunk(iv_ref, oa_ref):
        @plsc.parallel_loop(0, NV, 1, unroll=8)
        def _(i):
            iv = iv_ref[pl.ds(i * L, L)]
            oa_ref[pl.ds(i * L, L)] = plsc.load_gather(table_v, [iv])

    def gather_diff_chunk(iv_ref, oa_ref, ob_ref, h):
        m_base = h * (CE // K)

        def _body(r, carry):
            mi = lax.broadcast(m_base + r, (L,))
            cen = plsc.load_gather(cen_v, [mi])
            for j in range(KV):
                e = (r * KV + j) * L
                iv = iv_ref[pl.ds(e, L)]
                raw = plsc.load_gather(table_v, [iv])
                oa_ref[pl.ds(e, L)] = raw
                ob_ref[pl.ds(e, L)] = raw - cen
            return carry
        lax.fori_loop(0, CE // K, _body, 0, unroll=False)

    def run_task(ti, carry):
        t = wid + ti * NW

        @pl.when(t < NT)
        def _():
            b = t // CH
            ch = t - b * CH
            is_xyz = ch < 3

            def idx_src(h):
                return idx3.at[b, pl.ds(h * CE, CE)]

            @pl.when(jnp.logical_not(is_xyz))
            def _feat():
                tc = pltpu.async_copy(feat.at[b, ch - 3], table_v, sem_t)
                pltpu.async_copy(idx_src(0), idx_v[0], sem_i[0])
                tc.wait()
                for h in range(NCHUNK):
                    p = h % 2
                    if h + 1 < NCHUNK:
                        pltpu.async_copy(idx_src(h + 1), idx_v[(h + 1) % 2],
                                         sem_i[(h + 1) % 2])
                    pltpu.make_async_copy(idx_src(h), idx_v[p], sem_i[p]).wait()
                    if h >= 2:
                        pltpu.make_async_copy(
                            outa_v[p],
                            nf.at[b, ch, pl.ds((h - 2) * CE, CE)],
                            sem_o[p]).wait()
                    gather_chunk(idx_v[p], outa_v[p])
                    pltpu.async_copy(
                        outa_v[p], nf.at[b, ch, pl.ds(h * CE, CE)],
                        sem_o[p])
                for h in (NCHUNK - 2, NCHUNK - 1):
                    p = h % 2
                    pltpu.make_async_copy(
                        outa_v[p], nf.at[b, ch, pl.ds(h * CE, CE)],
                        sem_o[p]).wait()

            @pl.when(is_xyz)
            def _xyz():
                tc = pltpu.async_copy(xyz_t.at[b, ch], table_v, sem_t)
                cc = pltpu.async_copy(nxyz_t.at[b, ch], cen_v, sem_c)
                pltpu.async_copy(idx_src(0), idx_v[0], sem_i[0])
                tc.wait()
                cc.wait()
                for h in range(NCHUNK):
                    p = h % 2
                    if h + 1 < NCHUNK:
                        pltpu.async_copy(idx_src(h + 1), idx_v[(h + 1) % 2],
                                         sem_i[(h + 1) % 2])
                    pltpu.make_async_copy(idx_src(h), idx_v[p], sem_i[p]).wait()
                    if h >= 2:
                        pltpu.make_async_copy(
                            outa_v[p],
                            gx.at[b, ch, pl.ds((h - 2) * CE, CE)],
                            sem_o[p]).wait()
                        pltpu.make_async_copy(
                            outb_v[p],
                            nf.at[b, ch, pl.ds((h - 2) * CE, CE)],
                            sem_b[p]).wait()
                    gather_diff_chunk(idx_v[p], outa_v[p], outb_v[p], h)
                    pltpu.async_copy(
                        outa_v[p], gx.at[b, ch, pl.ds(h * CE, CE)],
                        sem_o[p])
                    pltpu.async_copy(
                        outb_v[p], nf.at[b, ch, pl.ds(h * CE, CE)],
                        sem_b[p])
                for h in (NCHUNK - 2, NCHUNK - 1):
                    p = h % 2
                    pltpu.make_async_copy(
                        outa_v[p], gx.at[b, ch, pl.ds(h * CE, CE)],
                        sem_o[p]).wait()
                    pltpu.make_async_copy(
                        outb_v[p], nf.at[b, ch, pl.ds(h * CE, CE)],
                        sem_b[p]).wait()

        return carry

    lax.fori_loop(0, TASKS_PER_W, run_task, 0, unroll=False)


@jax.jit
def kernel(xyz, new_xyz, features, idx):
    xyz_t = jnp.transpose(xyz, (0, 2, 1))
    nxyz_t = jnp.transpose(new_xyz, (0, 2, 1))
    idx3 = idx.astype(jnp.int32).reshape(B, MK)

    mesh = plsc.VectorSubcoreMesh(core_axis_name="c", subcore_axis_name="s")
    nf, gx = pl.kernel(
        _qag_kernel,
        out_type=(
            jax.ShapeDtypeStruct((B, CH, MK), jnp.float32),
            jax.ShapeDtypeStruct((B, 3, MK), jnp.float32),
        ),
        mesh=mesh,
        scratch_types=[
            pltpu.VMEM((N,), jnp.float32),
            pltpu.VMEM((CE,), jnp.int32),
            pltpu.VMEM((CE,), jnp.int32),
            pltpu.VMEM((CE,), jnp.float32),
            pltpu.VMEM((CE,), jnp.float32),
            pltpu.VMEM((CE,), jnp.float32),
            pltpu.VMEM((CE,), jnp.float32),
            pltpu.VMEM((M,), jnp.float32),
            pltpu.SemaphoreType.DMA,
            pltpu.SemaphoreType.DMA,
            pltpu.SemaphoreType.DMA,
            pltpu.SemaphoreType.DMA,
            pltpu.SemaphoreType.DMA,
            pltpu.SemaphoreType.DMA,
            pltpu.SemaphoreType.DMA,
            pltpu.SemaphoreType.DMA,
        ],
        compiler_params=pltpu.CompilerParams(needs_layout_passes=False),
    )(xyz_t, nxyz_t, features, idx3)

    new_features = nf.reshape(B, CH, M, K)
    grouped_xyz = gx.reshape(B, 3, M, K)
    return new_features, grouped_xyz
